# Initial kernel scaffold; baseline (speedup 1.0000x reference)
import jax
import jax.numpy as jnp
from jax.experimental import pallas as pl


def kernel(title_ids, title_token_ids, title_table, token_table):
    raise NotImplementedError("write your pallas kernel here")


# speedup vs baseline: 14.0051x; 14.0051x over previous
import functools

import jax
import jax.numpy as jnp
from jax import lax
from jax.experimental import pallas as pl
from jax.experimental.pallas import tpu as pltpu
from jax.experimental.pallas import tpu_sc as plsc

B = 16384
L = 20
DIM = 32
NC = 2
NS = 16
NW = NC * NS
NPW = B // NW
CB = 64
NCH = NPW // CB
GSL = 128
NG = CB * L // GSL


def _body(tids_flat, tok_flat, idst_flat, title_tab, token_tab, out_hbm,
          tidx_v, tok_idx_v, idst_v, trow_v, tok_rows_v, out64_v,
          row0_v, recip_v, n0_v,
          sem_idx0, sem_idx1, sem_g0, sem_g1, sem_out0, sem_out1, sem_r):
    wid = lax.axis_index("s") * NC + lax.axis_index("c")
    sem_idx = (sem_idx0, sem_idx1)
    sem_g = (sem_g0, sem_g1)
    sem_out = (sem_out0, sem_out1)

    def issue_stage1(i):
        buf = i % 2
        blk = wid * NCH + i
        return (
            pltpu.async_copy(tok_flat.at[pl.ds(blk * CB * L, CB * L)],
                             tok_idx_v.at[buf], sem_idx[buf]),
            pltpu.async_copy(tids_flat.at[pl.ds(blk * CB, CB)],
                             tidx_v.at[buf], sem_idx[buf]),
            pltpu.async_copy(idst_flat.at[pl.ds(blk * CB * L, CB * L)],
                             idst_v.at[buf], sem_idx[buf]),
        )

    def issue_gathers(i):
        buf = i % 2
        ds = []
        for j in range(NG):
            ds.append(pltpu.async_copy(
                token_tab.at[tok_idx_v.at[buf, pl.ds(j * GSL, GSL)]],
                tok_rows_v.at[buf, pl.ds(j * GSL, GSL)], sem_g[buf]))
        ds.append(pltpu.async_copy(
            title_tab.at[tidx_v.at[buf]], trow_v.at[buf], sem_g[buf]))
        return tuple(ds)

    d_row0 = pltpu.async_copy(token_tab.at[0], row0_v, sem_r)
    s1 = [None] * NCH
    gd = [None] * NCH
    od = [None] * NCH
    s1[0] = issue_stage1(0)
    for d in s1[0]:
        d.wait()
    gd[0] = issue_gathers(0)
    if NCH > 1:
        s1[1] = issue_stage1(1)
    d_row0.wait()
    r0a = row0_v[pl.ds(0, 16)]
    r0b = row0_v[pl.ds(16, 16)]

    for i in range(NCH):
        buf = i % 2
        for d in gd[i]:
            d.wait()
        if i + 1 < NCH:
            for d in s1[i + 1]:
                d.wait()
            gd[i + 1] = issue_gathers(i + 1)
        for g in range(CB // 16):
            cnt = jnp.zeros((16,), jnp.float32)
            for l in range(L):
                ids = idst_v[buf, pl.ds(l * CB + g * 16, 16)]
                cnt = cnt + (ids != 0).astype(jnp.float32)
            recip_v[pl.ds(g * 16, 16)] = 1.0 / jnp.maximum(cnt, 1.0)
            n0_v[pl.ds(g * 16, 16)] = jnp.float32(L) - cnt
        if i >= 2:
            od[i - 2].wait()

        def bbody(b, carry, buf=buf):
            r = b * L
            acc0 = tok_rows_v[buf, r, pl.ds(0, 16)]
            acc1 = tok_rows_v[buf, r, pl.ds(16, 16)]
            for t in range(1, L):
                acc0 = acc0 + tok_rows_v[buf, r + t, pl.ds(0, 16)]
                acc1 = acc1 + tok_rows_v[buf, r + t, pl.ds(16, 16)]
            bidx = jnp.broadcast_to(b, (16,)).astype(jnp.int32)
            rb = plsc.load_gather(recip_v, [bidx])
            n0b = plsc.load_gather(n0_v, [bidx])
            acc0 = (acc0 - n0b * r0a) * rb
            acc1 = (acc1 - n0b * r0b) * rb
            out64_v[buf, b, pl.ds(0, 16)] = trow_v[buf, b, pl.ds(0, 16)]
            out64_v[buf, b, pl.ds(16, 16)] = trow_v[buf, b, pl.ds(16, 16)]
            out64_v[buf, b, pl.ds(32, 16)] = acc0
            out64_v[buf, b, pl.ds(48, 16)] = acc1
            return carry

        lax.fori_loop(0, CB, bbody, 0)
        od[i] = pltpu.async_copy(
            out64_v.at[buf], out_hbm.at[pl.ds(wid * NPW + i * CB, CB)],
            sem_out[buf])
        if i + 2 < NCH:
            s1[i + 2] = issue_stage1(i + 2)

    od[NCH - 2].wait()
    od[NCH - 1].wait()


@functools.partial(jax.jit, static_argnames=())
def _launch(title_ids, title_token_ids, title_table, token_table):
    tids_flat = title_ids.astype(jnp.int32)
    tok_flat = title_token_ids.astype(jnp.int32).reshape(B * L)
    idst_flat = jnp.swapaxes(
        title_token_ids.astype(jnp.int32).reshape(B // CB, CB, L),
        1, 2).reshape(B * L)

    mesh = plsc.VectorSubcoreMesh(core_axis_name="c", subcore_axis_name="s",
                                  num_cores=NC, num_subcores=NS)
    f = pl.kernel(
        _body,
        out_type=jax.ShapeDtypeStruct((B, 2 * DIM), jnp.float32),
        mesh=mesh,
        scratch_types=[
            pltpu.VMEM((2, CB), jnp.int32),
            pltpu.VMEM((2, CB * L), jnp.int32),
            pltpu.VMEM((2, CB * L), jnp.int32),
            pltpu.VMEM((2, CB, DIM), jnp.float32),
            pltpu.VMEM((2, CB * L, DIM), jnp.float32),
            pltpu.VMEM((2, CB, 2 * DIM), jnp.float32),
            pltpu.VMEM((DIM,), jnp.float32),
            pltpu.VMEM((CB,), jnp.float32),
            pltpu.VMEM((CB,), jnp.float32),
            pltpu.SemaphoreType.DMA,
            pltpu.SemaphoreType.DMA,
            pltpu.SemaphoreType.DMA,
            pltpu.SemaphoreType.DMA,
            pltpu.SemaphoreType.DMA,
            pltpu.SemaphoreType.DMA,
            pltpu.SemaphoreType.DMA,
        ],
        compiler_params=pltpu.CompilerParams(needs_layout_passes=False,
                                             use_tc_tiling_on_sc=False),
    )
    return f(tids_flat, tok_flat, idst_flat, title_table, token_table)


def kernel(title_ids, title_token_ids, title_table, token_table):
    return _launch(title_ids, title_token_ids, title_table, token_table)


# speedup vs baseline: 14.8522x; 1.0605x over previous
import functools

import jax
import jax.numpy as jnp
from jax import lax
from jax.experimental import pallas as pl
from jax.experimental.pallas import tpu as pltpu
from jax.experimental.pallas import tpu_sc as plsc

B = 16384
L = 20
DIM = 32
NC = 2
NS = 16
NW = NC * NS
NPW = B // NW
CB = 64
NCH = NPW // CB
GSL = 128
NG = CB * L // GSL


def _body(tids_flat, tok_flat, title_tab, token_tab, out_hbm,
          tidx_v, tok_idx_v, trow_v, tok_rows_v, out64_v,
          row0_v, recip_v, n0_v,
          sem_idx0, sem_idx1, sem_g0, sem_g1, sem_out0, sem_out1, sem_r):
    wid = lax.axis_index("s") * NC + lax.axis_index("c")
    sem_idx = (sem_idx0, sem_idx1)
    sem_g = (sem_g0, sem_g1)
    sem_out = (sem_out0, sem_out1)

    def issue_stage1(i):
        buf = i % 2
        blk = wid * NCH + i
        return (
            pltpu.async_copy(tok_flat.at[pl.ds(blk * CB * L, CB * L)],
                             tok_idx_v.at[buf], sem_idx[buf]),
            pltpu.async_copy(tids_flat.at[pl.ds(blk * CB, CB)],
                             tidx_v.at[buf], sem_idx[buf]),
        )

    def issue_gathers(i):
        buf = i % 2
        ds = []
        for j in range(NG):
            ds.append(pltpu.async_copy(
                token_tab.at[tok_idx_v.at[buf, pl.ds(j * GSL, GSL)]],
                tok_rows_v.at[buf, pl.ds(j * GSL, GSL)], sem_g[buf]))
        ds.append(pltpu.async_copy(
            title_tab.at[tidx_v.at[buf]], trow_v.at[buf], sem_g[buf]))
        return tuple(ds)

    d_row0 = pltpu.async_copy(token_tab.at[0], row0_v, sem_r)
    s1 = [None] * NCH
    gd = [None] * NCH
    od = [None] * NCH
    s1[0] = issue_stage1(0)
    for d in s1[0]:
        d.wait()
    gd[0] = issue_gathers(0)
    if NCH > 1:
        s1[1] = issue_stage1(1)
    d_row0.wait()
    r0a = row0_v[pl.ds(0, 16)]
    r0b = row0_v[pl.ds(16, 16)]

    for i in range(NCH):
        buf = i % 2
        for d in gd[i]:
            d.wait()
        if i + 1 < NCH:
            for d in s1[i + 1]:
                d.wait()
            gd[i + 1] = issue_gathers(i + 1)
        for g in range(CB // 16):
            pvec = (lax.iota(jnp.int32, 16) + g * 16) * L
            cnti = jnp.zeros((16,), jnp.int32)
            for t in range(L):
                ids = plsc.load_gather(tok_idx_v.at[buf], [pvec + t])
                cnti = cnti + jnp.minimum(ids, 1)
            cnt = cnti.astype(jnp.float32)
            recip_v[pl.ds(g * 16, 16)] = 1.0 / jnp.maximum(cnt, 1.0)
            n0_v[pl.ds(g * 16, 16)] = jnp.float32(L) - cnt
        if i >= 2:
            od[i - 2].wait()

        def bbody(b, carry, buf=buf):
            r = b * L
            acc0 = tok_rows_v[buf, r, pl.ds(0, 16)]
            acc1 = tok_rows_v[buf, r, pl.ds(16, 16)]
            for t in range(1, L):
                acc0 = acc0 + tok_rows_v[buf, r + t, pl.ds(0, 16)]
                acc1 = acc1 + tok_rows_v[buf, r + t, pl.ds(16, 16)]
            bidx = jnp.broadcast_to(b, (16,)).astype(jnp.int32)
            rb = plsc.load_gather(recip_v, [bidx])
            n0b = plsc.load_gather(n0_v, [bidx])
            acc0 = (acc0 - n0b * r0a) * rb
            acc1 = (acc1 - n0b * r0b) * rb
            out64_v[buf, b, pl.ds(0, 16)] = trow_v[buf, b, pl.ds(0, 16)]
            out64_v[buf, b, pl.ds(16, 16)] = trow_v[buf, b, pl.ds(16, 16)]
            out64_v[buf, b, pl.ds(32, 16)] = acc0
            out64_v[buf, b, pl.ds(48, 16)] = acc1
            return carry

        lax.fori_loop(0, CB, bbody, 0)
        od[i] = pltpu.async_copy(
            out64_v.at[buf], out_hbm.at[pl.ds(wid * NPW + i * CB, CB)],
            sem_out[buf])
        if i + 2 < NCH:
            s1[i + 2] = issue_stage1(i + 2)

    od[NCH - 2].wait()
    od[NCH - 1].wait()


@functools.partial(jax.jit, static_argnames=())
def _launch(title_ids, title_token_ids, title_table, token_table):
    tids_flat = title_ids.astype(jnp.int32)
    tok_flat = title_token_ids.astype(jnp.int32).reshape(B * L)

    mesh = plsc.VectorSubcoreMesh(core_axis_name="c", subcore_axis_name="s",
                                  num_cores=NC, num_subcores=NS)
    f = pl.kernel(
        _body,
        out_type=jax.ShapeDtypeStruct((B, 2 * DIM), jnp.float32),
        mesh=mesh,
        scratch_types=[
            pltpu.VMEM((2, CB), jnp.int32),
            pltpu.VMEM((2, CB * L), jnp.int32),
            pltpu.VMEM((2, CB, DIM), jnp.float32),
            pltpu.VMEM((2, CB * L, DIM), jnp.float32),
            pltpu.VMEM((2, CB, 2 * DIM), jnp.float32),
            pltpu.VMEM((DIM,), jnp.float32),
            pltpu.VMEM((CB,), jnp.float32),
            pltpu.VMEM((CB,), jnp.float32),
            pltpu.SemaphoreType.DMA,
            pltpu.SemaphoreType.DMA,
            pltpu.SemaphoreType.DMA,
            pltpu.SemaphoreType.DMA,
            pltpu.SemaphoreType.DMA,
            pltpu.SemaphoreType.DMA,
            pltpu.SemaphoreType.DMA,
        ],
        compiler_params=pltpu.CompilerParams(needs_layout_passes=False,
                                             use_tc_tiling_on_sc=False),
    )
    return f(tids_flat, tok_flat, title_table, token_table)


def kernel(title_ids, title_token_ids, title_table, token_table):
    return _launch(title_ids, title_token_ids, title_table, token_table)


# speedup vs baseline: 16.3637x; 1.1018x over previous
import functools

import jax
import jax.numpy as jnp
from jax import lax
from jax.experimental import pallas as pl
from jax.experimental.pallas import tpu as pltpu
from jax.experimental.pallas import tpu_sc as plsc

B = 16384
L = 20
DIM = 32
TPAD = 128
NC = 2
NS = 16
NW = NC * NS
NPW = B // NW
CB = 64
NCH = NPW // CB


def _body(tids_flat, ids_t, title_pad, token_tab, out_hbm,
          tidx_v, tok_idx_v, trow_v, tok_rows_v, out64_v,
          row0_v, recip_v, n0_v,
          sem_idx0, sem_idx1, sem_g0, sem_g1, sem_out0, sem_out1, sem_r):
    wid = lax.axis_index("s") * NC + lax.axis_index("c")
    sem_idx = (sem_idx0, sem_idx1)
    sem_g = (sem_g0, sem_g1)
    sem_out = (sem_out0, sem_out1)

    def issue_stage1(i):
        buf = i % 2
        boff = wid * NPW + i * CB
        return (
            pltpu.async_copy(ids_t.at[:, pl.ds(boff, CB)],
                             tok_idx_v.at[buf], sem_idx[buf]),
            pltpu.async_copy(tids_flat.at[pl.ds(boff, CB)],
                             tidx_v.at[buf], sem_idx[buf]),
        )

    def issue_gathers(i):
        buf = i % 2
        ds = []
        for l in range(L):
            ds.append(pltpu.async_copy(
                token_tab.at[tok_idx_v.at[buf, l]],
                tok_rows_v.at[buf, pl.ds(l * CB, CB)], sem_g[buf]))
        ds.append(pltpu.async_copy(
            title_pad.at[tidx_v.at[buf]], trow_v.at[buf], sem_g[buf]))
        return tuple(ds)

    d_row0 = pltpu.async_copy(token_tab.at[0], row0_v, sem_r)
    s1 = [None] * NCH
    gd = [None] * NCH
    od = [None] * NCH
    s1[0] = issue_stage1(0)
    for d in s1[0]:
        d.wait()
    gd[0] = issue_gathers(0)
    if NCH > 1:
        s1[1] = issue_stage1(1)
    d_row0.wait()
    r0a = row0_v[pl.ds(0, 16)]
    r0b = row0_v[pl.ds(16, 16)]

    for i in range(NCH):
        buf = i % 2
        for d in gd[i]:
            d.wait()
        if i + 1 < NCH:
            for d in s1[i + 1]:
                d.wait()
            gd[i + 1] = issue_gathers(i + 1)
        for g in range(CB // 16):
            cnti = jnp.zeros((16,), jnp.int32)
            for l in range(L):
                ids = tok_idx_v[buf, l, pl.ds(g * 16, 16)]
                cnti = cnti + jnp.minimum(ids, 1)
            cnt = cnti.astype(jnp.float32)
            recip_v[pl.ds(g * 16, 16)] = 1.0 / jnp.maximum(cnt, 1.0)
            n0_v[pl.ds(g * 16, 16)] = jnp.float32(L) - cnt
        if i >= 2:
            od[i - 2].wait()

        def bbody(b, carry, buf=buf):
            acc0 = tok_rows_v[buf, b, pl.ds(0, 16)]
            acc1 = tok_rows_v[buf, b, pl.ds(16, 16)]
            for l in range(1, L):
                acc0 = acc0 + tok_rows_v[buf, l * CB + b, pl.ds(0, 16)]
                acc1 = acc1 + tok_rows_v[buf, l * CB + b, pl.ds(16, 16)]
            bidx = jnp.broadcast_to(b, (16,)).astype(jnp.int32)
            rb = plsc.load_gather(recip_v, [bidx])
            n0b = plsc.load_gather(n0_v, [bidx])
            acc0 = (acc0 - n0b * r0a) * rb
            acc1 = (acc1 - n0b * r0b) * rb
            out64_v[buf, b, pl.ds(0, 16)] = trow_v[buf, b, pl.ds(0, 16)]
            out64_v[buf, b, pl.ds(16, 16)] = trow_v[buf, b, pl.ds(16, 16)]
            out64_v[buf, b, pl.ds(32, 16)] = acc0
            out64_v[buf, b, pl.ds(48, 16)] = acc1
            return carry

        lax.fori_loop(0, CB, bbody, 0)
        od[i] = pltpu.async_copy(
            out64_v.at[buf], out_hbm.at[pl.ds(wid * NPW + i * CB, CB)],
            sem_out[buf])
        if i + 2 < NCH:
            s1[i + 2] = issue_stage1(i + 2)

    od[NCH - 2].wait()
    od[NCH - 1].wait()


@functools.partial(jax.jit, static_argnames=())
def _launch(title_ids, title_token_ids, title_table, token_table):
    tids_flat = title_ids.astype(jnp.int32)
    ids_t = title_token_ids.astype(jnp.int32).T
    title_pad = jnp.pad(title_table, ((0, 0), (0, TPAD - DIM)))

    mesh = plsc.VectorSubcoreMesh(core_axis_name="c", subcore_axis_name="s",
                                  num_cores=NC, num_subcores=NS)
    f = pl.kernel(
        _body,
        out_type=jax.ShapeDtypeStruct((B, 2 * DIM), jnp.float32),
        mesh=mesh,
        scratch_types=[
            pltpu.VMEM((2, CB), jnp.int32),
            pltpu.VMEM((2, L, CB), jnp.int32),
            pltpu.VMEM((2, CB, TPAD), jnp.float32),
            pltpu.VMEM((2, CB * L, DIM), jnp.float32),
            pltpu.VMEM((2, CB, 2 * DIM), jnp.float32),
            pltpu.VMEM((DIM,), jnp.float32),
            pltpu.VMEM((CB,), jnp.float32),
            pltpu.VMEM((CB,), jnp.float32),
            pltpu.SemaphoreType.DMA,
            pltpu.SemaphoreType.DMA,
            pltpu.SemaphoreType.DMA,
            pltpu.SemaphoreType.DMA,
            pltpu.SemaphoreType.DMA,
            pltpu.SemaphoreType.DMA,
            pltpu.SemaphoreType.DMA,
        ],
        compiler_params=pltpu.CompilerParams(needs_layout_passes=False,
                                             use_tc_tiling_on_sc=False),
    )
    return f(tids_flat, ids_t, title_pad, token_table)


def kernel(title_ids, title_token_ids, title_table, token_table):
    return _launch(title_ids, title_token_ids, title_table, token_table)


# speedup vs baseline: 16.4463x; 1.0050x over previous
import functools

import jax
import jax.numpy as jnp
from jax import lax
from jax.experimental import pallas as pl
from jax.experimental.pallas import tpu as pltpu
from jax.experimental.pallas import tpu_sc as plsc

B = 16384
L = 20
DIM = 32
MAX_TOKENS = 10000
TPAD = 128
NC = 2
NS = 16
NW = NC * NS
NPW = B // NW
CB = 64
NCH = NPW // CB


def _body(tids_flat, ids_t, title_pad, token_tab, out_hbm,
          tidx_v, tok_idx_v, trow_v, tok_rows_v, out64_v,
          row0_v, recip_v, n0_v,
          sem_idx0, sem_idx1, sem_g0, sem_g1, sem_out0, sem_out1, sem_r):
    wid = lax.axis_index("s") * NC + lax.axis_index("c")
    sem_idx = (sem_idx0, sem_idx1)
    sem_g = (sem_g0, sem_g1)
    sem_out = (sem_out0, sem_out1)

    def issue_stage1(i):
        buf = i % 2
        boff = wid * NPW + i * CB
        return (
            pltpu.async_copy(ids_t.at[:, pl.ds(boff, CB)],
                             tok_idx_v.at[buf], sem_idx[buf]),
            pltpu.async_copy(tids_flat.at[pl.ds(boff, CB)],
                             tidx_v.at[buf], sem_idx[buf]),
        )

    def issue_gathers(i):
        buf = i % 2
        ds = []
        for l in range(L):
            ds.append(pltpu.async_copy(
                token_tab.at[tok_idx_v.at[buf, l]],
                tok_rows_v.at[buf, pl.ds(l * CB, CB)], sem_g[buf]))
        ds.append(pltpu.async_copy(
            title_pad.at[tidx_v.at[buf]], trow_v.at[buf], sem_g[buf]))
        return tuple(ds)

    d_row0 = pltpu.async_copy(token_tab.at[0], row0_v, sem_r)
    s1 = [None] * NCH
    gd = [None] * NCH
    od = [None] * NCH
    s1[0] = issue_stage1(0)
    for d in s1[0]:
        d.wait()
    gd[0] = issue_gathers(0)
    if NCH > 1:
        s1[1] = issue_stage1(1)
    d_row0.wait()
    r0a = row0_v[pl.ds(0, 16)]
    r0b = row0_v[pl.ds(16, 16)]

    for i in range(NCH):
        buf = i % 2
        for d in gd[i]:
            d.wait()
        if i + 1 < NCH:
            for d in s1[i + 1]:
                d.wait()
            gd[i + 1] = issue_gathers(i + 1)
        for g in range(CB // 16):
            cnti = jnp.zeros((16,), jnp.int32)
            for l in range(L):
                ids = tok_idx_v[buf, l, pl.ds(g * 16, 16)]
                cnti = cnti + jnp.minimum(ids, 1)
            cnt = cnti.astype(jnp.float32)
            recip_v[pl.ds(g * 16, 16)] = 1.0 / jnp.maximum(cnt, 1.0)
            n0_v[pl.ds(g * 16, 16)] = jnp.float32(L) - cnt
        if i >= 2:
            od[i - 2].wait()

        def bbody(b, carry, buf=buf):
            acc0 = tok_rows_v[buf, b, pl.ds(0, 16)]
            acc1 = tok_rows_v[buf, b, pl.ds(16, 16)]
            for l in range(1, L):
                acc0 = acc0 + tok_rows_v[buf, l * CB + b, pl.ds(0, 16)]
                acc1 = acc1 + tok_rows_v[buf, l * CB + b, pl.ds(16, 16)]
            bidx = jnp.broadcast_to(b, (16,)).astype(jnp.int32)
            rb = plsc.load_gather(recip_v, [bidx])
            n0b = plsc.load_gather(n0_v, [bidx])
            acc0 = (acc0 - n0b * r0a) * rb
            acc1 = (acc1 - n0b * r0b) * rb
            o = b * 2 * DIM
            out64_v[buf, pl.ds(o, 16)] = trow_v[buf, b, pl.ds(0, 16)]
            out64_v[buf, pl.ds(o + 16, 16)] = trow_v[buf, b, pl.ds(16, 16)]
            out64_v[buf, pl.ds(o + 32, 16)] = acc0
            out64_v[buf, pl.ds(o + 48, 16)] = acc1
            return carry

        lax.fori_loop(0, CB, bbody, 0)
        od[i] = pltpu.async_copy(
            out64_v.at[buf],
            out_hbm.at[pl.ds((wid * NPW + i * CB) * 2 * DIM, CB * 2 * DIM)],
            sem_out[buf])
        if i + 2 < NCH:
            s1[i + 2] = issue_stage1(i + 2)

    od[NCH - 2].wait()
    od[NCH - 1].wait()


@functools.partial(jax.jit, static_argnames=())
def _launch(title_ids, title_token_ids, title_table, token_table):
    tids_flat = title_ids.astype(jnp.int32)
    ids_lin, tok_lin = lax.optimization_barrier(
        (title_token_ids.astype(jnp.int32).T.reshape(B * L),
         token_table.reshape(MAX_TOKENS * DIM)))
    ids_t = ids_lin.reshape(L, B)
    token_lin2d = tok_lin.reshape(MAX_TOKENS, DIM)
    title_pad = jnp.pad(title_table, ((0, 0), (0, TPAD - DIM)))

    mesh = plsc.VectorSubcoreMesh(core_axis_name="c", subcore_axis_name="s",
                                  num_cores=NC, num_subcores=NS)
    f = pl.kernel(
        _body,
        out_type=jax.ShapeDtypeStruct((B * 2 * DIM,), jnp.float32),
        mesh=mesh,
        scratch_types=[
            pltpu.VMEM((2, CB), jnp.int32),
            pltpu.VMEM((2, L, CB), jnp.int32),
            pltpu.VMEM((2, CB, TPAD), jnp.float32),
            pltpu.VMEM((2, CB * L, DIM), jnp.float32),
            pltpu.VMEM((2, CB * 2 * DIM), jnp.float32),
            pltpu.VMEM((DIM,), jnp.float32),
            pltpu.VMEM((CB,), jnp.float32),
            pltpu.VMEM((CB,), jnp.float32),
            pltpu.SemaphoreType.DMA,
            pltpu.SemaphoreType.DMA,
            pltpu.SemaphoreType.DMA,
            pltpu.SemaphoreType.DMA,
            pltpu.SemaphoreType.DMA,
            pltpu.SemaphoreType.DMA,
            pltpu.SemaphoreType.DMA,
        ],
        compiler_params=pltpu.CompilerParams(needs_layout_passes=False,
                                             use_tc_tiling_on_sc=False),
    )
    return f(tids_flat, ids_t, title_pad, token_lin2d).reshape(B, 2 * DIM)


def kernel(title_ids, title_token_ids, title_table, token_table):
    return _launch(title_ids, title_token_ids, title_table, token_table)
